# Initial kernel scaffold; baseline (speedup 1.0000x reference)
import jax
import jax.numpy as jnp
from jax.experimental import pallas as pl


def kernel(x, Wg, bg, W1, b1, W2, b2):
    raise NotImplementedError("write your pallas kernel here")


# speedup vs baseline: 1.6141x; 1.6141x over previous
import functools

import jax
import jax.numpy as jnp
from jax.experimental import pallas as pl
from jax.experimental.pallas import tpu as pltpu

E = 8
TOP_K = 2
D_IN = 1024
D_OUT = 1024
D_PROJ = 256
N_TOK = 4096

BM = 512
LANES = 128

_NEG = -1e30


def _gelu_tanh(x):
    return 0.5 * x * (1.0 + jnp.tanh(jnp.sqrt(2.0 / jnp.pi) * (x + 0.044715 * x ** 3)))


def _moe_kernel(x_ref, wg_ref, bg_ref, w1_ref, b1_ref, w2_ref, b2_ref,
                out_ref, wfull_ref):
    e = pl.program_id(1)
    xb = x_ref[...]
    lane = jax.lax.broadcasted_iota(jnp.int32, (BM, LANES), 1)

    @pl.when(e == 0)
    def _gate():
        logits = (jnp.dot(xb, wg_ref[...], preferred_element_type=jnp.float32)
                  + bg_ref[...]) * (1.0 / jnp.sqrt(jnp.float32(D_IN)))
        logits = jnp.where(lane < E, logits, _NEG)
        m1 = jnp.max(logits, axis=1, keepdims=True)
        p = jnp.exp(logits - m1)
        probs = p / jnp.sum(p, axis=1, keepdims=True)
        i1 = jnp.min(jnp.where(logits >= m1, lane, LANES), axis=1, keepdims=True)
        logits2 = jnp.where(lane == i1, _NEG, logits)
        m2 = jnp.max(logits2, axis=1, keepdims=True)
        i2 = jnp.min(jnp.where(logits2 >= m2, lane, LANES), axis=1, keepdims=True)
        wfull_ref[...] = probs * ((lane == i1) | (lane == i2)).astype(jnp.float32)

    w_col = jnp.sum(
        wfull_ref[...] * (lane == e).astype(jnp.float32), axis=1, keepdims=True)
    h = _gelu_tanh(
        jnp.dot(xb, w1_ref[0], preferred_element_type=jnp.float32) + b1_ref[0])
    y = jnp.dot(h, w2_ref[0], preferred_element_type=jnp.float32) + b2_ref[0]
    contrib = w_col * y

    @pl.when(e == 0)
    def _init():
        out_ref[...] = contrib

    @pl.when(e != 0)
    def _acc():
        out_ref[...] += contrib


@jax.jit
def kernel(x, Wg, bg, W1, b1, W2, b2):
    in_shape = x.shape
    xf = x.reshape(-1, D_IN)
    n = xf.shape[0]
    wg_pad = jnp.pad(Wg, ((0, 0), (0, LANES - E)))
    bg_pad = jnp.pad(bg, (0, LANES - E)).reshape(1, LANES)
    b1r = b1.reshape(E, 1, D_PROJ)
    b2r = b2.reshape(E, 1, D_OUT)
    grid = (n // BM, E)
    y = pl.pallas_call(
        _moe_kernel,
        grid=grid,
        in_specs=[
            pl.BlockSpec((BM, D_IN), lambda i, e: (i, 0)),
            pl.BlockSpec((D_IN, LANES), lambda i, e: (0, 0)),
            pl.BlockSpec((1, LANES), lambda i, e: (0, 0)),
            pl.BlockSpec((1, D_IN, D_PROJ), lambda i, e: (e, 0, 0)),
            pl.BlockSpec((1, 1, D_PROJ), lambda i, e: (e, 0, 0)),
            pl.BlockSpec((1, D_PROJ, D_OUT), lambda i, e: (e, 0, 0)),
            pl.BlockSpec((1, 1, D_OUT), lambda i, e: (e, 0, 0)),
        ],
        out_specs=pl.BlockSpec((BM, D_OUT), lambda i, e: (i, 0)),
        out_shape=jax.ShapeDtypeStruct((n, D_OUT), jnp.float32),
        scratch_shapes=[pltpu.VMEM((BM, LANES), jnp.float32)],
        compiler_params=pltpu.CompilerParams(
            dimension_semantics=("parallel", "arbitrary")),
    )(xf, wg_pad, bg_pad, W1, b1r, W2, b2r)
    return y.reshape(in_shape[:-1] + (D_OUT,))
